# baseline (speedup 1.0000x reference)
import functools

import jax
import jax.numpy as jnp
from jax import lax
from jax.experimental import pallas as pl
from jax.experimental.pallas import tpu as pltpu
from jax.experimental.pallas import tpu_sc as plsc

K = 256
D = 8
SUB = 8
EMB = 64

_NC = 2
_NS = 16
_NW = _NC * _NS

_R = 128


def _sc_gather(idx, table, n_tokens):
    per_w = n_tokens // _NW
    n_chunks = per_w // _R

    mesh = plsc.VectorSubcoreMesh(
        core_axis_name="c", subcore_axis_name="s",
        num_cores=_NC, num_subcores=_NS)

    @functools.partial(
        pl.kernel,
        mesh=mesh,
        out_type=jax.ShapeDtypeStruct((n_tokens, EMB), table.dtype),
        scratch_types=[
            pltpu.VMEM((_R,), jnp.int32),
            pltpu.VMEM((_R, EMB), table.dtype),
            pltpu.SemaphoreType.DMA,
        ],
        compiler_params=pltpu.CompilerParams(use_tc_tiling_on_sc=False),
    )
    def gather_kernel(idx_hbm, table_hbm, out_hbm, idx_v, rows_v, sem):
        wid = lax.axis_index("s") * _NC + lax.axis_index("c")
        base = wid * per_w
        for t in range(n_chunks):
            off = base + t * _R
            pltpu.sync_copy(idx_hbm.at[pl.ds(off, _R)], idx_v)
            pltpu.async_copy(table_hbm.at[idx_v], rows_v, sem).wait()
            pltpu.sync_copy(rows_v, out_hbm.at[pl.ds(off, _R)])

    return gather_kernel(idx, table)


def _tc_block(x_ref, waug_ref, cnorm_ref, w2t_ref, out_ref):
    x = x_ref[...]
    blk = x.shape[0]
    xt = jnp.transpose(x)
    dot2 = lax.dot_general(
        waug_ref[...], xt, (((1,), (0,)), ((), ())),
        preferred_element_type=jnp.float32)
    st = dot2 - cnorm_ref[...]
    onehots = []
    k_iota = lax.broadcasted_iota(jnp.int32, (K, blk), 0)
    for d in range(D):
        s_d = st[d * K:(d + 1) * K, :]
        m_d = jnp.max(s_d, axis=0, keepdims=True)
        cand = jnp.where(s_d == m_d, k_iota, K)
        kmin = jnp.min(cand, axis=0, keepdims=True)
        onehots.append((cand == kmin).astype(jnp.bfloat16))
    onehot = jnp.concatenate(onehots, axis=0)
    out_t = lax.dot_general(
        w2t_ref[...], onehot, (((1,), (0,)), ((), ())),
        preferred_element_type=jnp.float32)
    out_ref[...] = jnp.transpose(out_t)


def _tc_compute(x, centroids, n_tokens, blk):
    c = centroids
    eye = jnp.eye(D, dtype=jnp.float32)
    waug = jnp.einsum(
        "de,eks->ekds", eye, 2.0 * c).reshape(D * K, EMB).astype(jnp.bfloat16)
    cnorm = jnp.sum(c * c, axis=-1).reshape(D * K, 1)
    w2t = jnp.einsum("de,dks->esdk", eye, c).reshape(EMB, D * K)
    w2t = w2t.astype(jnp.bfloat16)

    grid = (n_tokens // blk,)
    return pl.pallas_call(
        _tc_block,
        grid=grid,
        in_specs=[
            pl.BlockSpec((blk, EMB), lambda i: (i, 0)),
            pl.BlockSpec((D * K, EMB), lambda i: (0, 0)),
            pl.BlockSpec((D * K, 1), lambda i: (0, 0)),
            pl.BlockSpec((EMB, D * K), lambda i: (0, 0)),
        ],
        out_specs=pl.BlockSpec((blk, EMB), lambda i: (i, 0)),
        out_shape=jax.ShapeDtypeStruct((n_tokens, EMB), jnp.float32),
    )(x, waug, cnorm, w2t)


def kernel(inputs, emb_table, centroids):
    inputs = inputs.astype(jnp.int32)
    idx = inputs.reshape(-1)
    n_tokens = idx.shape[0]
    x = _sc_gather(idx, emb_table.astype(jnp.bfloat16), n_tokens)
    out = _tc_compute(x, centroids, n_tokens, blk=512)
    return out.reshape(inputs.shape + (EMB,))


# speedup vs baseline: 1.2871x; 1.2871x over previous
import functools

import jax
import jax.numpy as jnp
from jax import lax
from jax.experimental import pallas as pl
from jax.experimental.pallas import tpu as pltpu
from jax.experimental.pallas import tpu_sc as plsc

K = 256
D = 8
SUB = 8
EMB = 64

_NC = 2
_NS = 16
_NW = _NC * _NS

_R = 128


def _sc_gather(idx, table, n_tokens):
    per_w = n_tokens // _NW
    n_chunks = per_w // _R

    mesh = plsc.VectorSubcoreMesh(
        core_axis_name="c", subcore_axis_name="s",
        num_cores=_NC, num_subcores=_NS)

    @functools.partial(
        pl.kernel,
        mesh=mesh,
        out_type=jax.ShapeDtypeStruct((n_tokens, EMB), table.dtype),
        scratch_types=[
            pltpu.VMEM((_R,), jnp.int32),
            pltpu.VMEM((_R, EMB), table.dtype),
            pltpu.SemaphoreType.DMA,
        ],
        compiler_params=pltpu.CompilerParams(use_tc_tiling_on_sc=False),
    )
    def gather_kernel(idx_hbm, table_hbm, out_hbm, idx_v, rows_v, sem):
        wid = lax.axis_index("s") * _NC + lax.axis_index("c")
        base = wid * per_w
        for t in range(n_chunks):
            off = base + t * _R
            pltpu.sync_copy(idx_hbm.at[pl.ds(off, _R)], idx_v)
            pltpu.async_copy(table_hbm.at[idx_v], rows_v, sem).wait()
            pltpu.sync_copy(rows_v, out_hbm.at[pl.ds(off, _R)])

    return gather_kernel(idx, table)


def _tc_block(x_ref, waug_ref, cnorm_ref, w2t_ref, out_ref):
    x = x_ref[...]
    blk = x.shape[0]
    xt = jnp.transpose(x.astype(jnp.bfloat16))
    dot2 = lax.dot_general(
        waug_ref[...], xt, (((1,), (0,)), ((), ())),
        preferred_element_type=jnp.float32)
    st = dot2 - cnorm_ref[...]
    onehots = []
    k_iota = lax.broadcasted_iota(jnp.int32, (K, blk), 0)
    for d in range(D):
        s_d = st[d * K:(d + 1) * K, :]
        m_d = jnp.max(s_d, axis=0, keepdims=True)
        cand = jnp.where(s_d == m_d, k_iota, K)
        kmin = jnp.min(cand, axis=0, keepdims=True)
        onehots.append((cand == kmin).astype(jnp.bfloat16))
    onehot = jnp.concatenate(onehots, axis=0)
    out_t = lax.dot_general(
        w2t_ref[...], onehot, (((1,), (0,)), ((), ())),
        preferred_element_type=jnp.float32)
    r0, r1, r2 = out_ref.shape
    out_ref[...] = jnp.transpose(out_t).reshape(r0, r1, r2)


def _tc_compute(x, centroids, ishape, blk):
    n_tokens = ishape[0] * ishape[1]
    c = centroids
    eye = jnp.eye(D, dtype=jnp.float32)
    waug = jnp.einsum(
        "de,eks->ekds", eye, 2.0 * c).reshape(D * K, EMB).astype(jnp.bfloat16)
    cnorm = jnp.sum(c * c, axis=-1).reshape(D * K, 1)
    w2t = jnp.einsum("de,dks->esdk", eye, c).reshape(EMB, D * K)
    w2t = w2t.astype(jnp.bfloat16)

    rows = blk // ishape[1]
    grid = (n_tokens // blk,)
    return pl.pallas_call(
        _tc_block,
        grid=grid,
        in_specs=[
            pl.BlockSpec((blk, EMB), lambda i: (i, 0)),
            pl.BlockSpec((D * K, EMB), lambda i: (0, 0)),
            pl.BlockSpec((D * K, 1), lambda i: (0, 0)),
            pl.BlockSpec((EMB, D * K), lambda i: (0, 0)),
        ],
        out_specs=pl.BlockSpec((rows, ishape[1], EMB), lambda i: (i, 0, 0)),
        out_shape=jax.ShapeDtypeStruct((*ishape, EMB), jnp.float32),
    )(x, waug, cnorm, w2t)


def kernel(inputs, emb_table, centroids):
    inputs = inputs.astype(jnp.int32)
    idx = inputs.reshape(-1)
    n_tokens = idx.shape[0]
    x = _sc_gather(idx, emb_table, n_tokens)
    return _tc_compute(x, centroids, inputs.shape, blk=640)


# speedup vs baseline: 1.3180x; 1.0241x over previous
import functools

import jax
import jax.numpy as jnp
from jax import lax
from jax.experimental import pallas as pl
from jax.experimental.pallas import tpu as pltpu
from jax.experimental.pallas import tpu_sc as plsc

K = 256
D = 8
SUB = 8
EMB = 64

_NC = 2
_NS = 16
_NW = _NC * _NS

_R = 128


def _sc_gather(idx, table, n_tokens):
    per_w = n_tokens // _NW
    n_chunks = per_w // _R

    mesh = plsc.VectorSubcoreMesh(
        core_axis_name="c", subcore_axis_name="s",
        num_cores=_NC, num_subcores=_NS)

    @functools.partial(
        pl.kernel,
        mesh=mesh,
        out_type=jax.ShapeDtypeStruct((n_tokens, EMB), table.dtype),
        scratch_types=[
            pltpu.VMEM((_R,), jnp.int32),
            pltpu.VMEM((_R, EMB), table.dtype),
            pltpu.SemaphoreType.DMA,
        ],
        compiler_params=pltpu.CompilerParams(use_tc_tiling_on_sc=False),
    )
    def gather_kernel(idx_hbm, table_hbm, out_hbm, idx_v, rows_v, sem):
        wid = lax.axis_index("s") * _NC + lax.axis_index("c")
        base = wid * per_w
        for t in range(n_chunks):
            off = base + t * _R
            pltpu.sync_copy(idx_hbm.at[pl.ds(off, _R)], idx_v)
            pltpu.async_copy(table_hbm.at[idx_v], rows_v, sem).wait()
            pltpu.sync_copy(rows_v, out_hbm.at[pl.ds(off, _R)])

    return gather_kernel(idx, table)


def _tc_block(x_ref, waug_ref, cnorm_ref, w2t_ref, out_ref):
    x = x_ref[...]
    blk = x.shape[0]
    xt = jnp.transpose(x.astype(jnp.bfloat16))
    dot2 = lax.dot_general(
        waug_ref[...], xt, (((1,), (0,)), ((), ())),
        preferred_element_type=jnp.float32)
    st = dot2 - cnorm_ref[...]
    onehots = []
    k_iota = lax.broadcasted_iota(jnp.int32, (K, blk), 0)
    for d in range(D):
        s_d = st[d * K:(d + 1) * K, :]
        m_d = jnp.max(s_d, axis=0, keepdims=True)
        cand = jnp.where(s_d == m_d, k_iota, K)
        kmin = jnp.min(cand, axis=0, keepdims=True)
        onehots.append((cand == kmin).astype(jnp.bfloat16))
    onehot = jnp.concatenate(onehots, axis=0)
    out_t = lax.dot_general(
        w2t_ref[...], onehot, (((1,), (0,)), ((), ())),
        preferred_element_type=jnp.float32)
    r0, r1, r2 = out_ref.shape
    out_ref[...] = jnp.transpose(out_t).reshape(r0, r1, r2)


def _tc_compute(x, centroids, ishape, blk):
    n_tokens = ishape[0] * ishape[1]
    c = centroids
    eye = jnp.eye(D, dtype=jnp.float32)
    waug = jnp.einsum(
        "de,eks->ekds", eye, 2.0 * c).reshape(D * K, EMB).astype(jnp.bfloat16)
    cnorm = jnp.sum(c * c, axis=-1).reshape(D * K, 1)
    w2t = jnp.einsum("de,dks->esdk", eye, c).reshape(EMB, D * K)
    w2t = w2t.astype(jnp.bfloat16)

    rows = blk // ishape[1]
    grid = (n_tokens // blk,)
    return pl.pallas_call(
        _tc_block,
        grid=grid,
        in_specs=[
            pl.BlockSpec((blk, EMB), lambda i: (i, 0)),
            pl.BlockSpec((D * K, EMB), lambda i: (0, 0)),
            pl.BlockSpec((D * K, 1), lambda i: (0, 0)),
            pl.BlockSpec((EMB, D * K), lambda i: (0, 0)),
        ],
        out_specs=pl.BlockSpec((rows, ishape[1], EMB), lambda i: (i, 0, 0)),
        out_shape=jax.ShapeDtypeStruct((*ishape, EMB), jnp.float32),
    )(x, waug, cnorm, w2t)


def kernel(inputs, emb_table, centroids):
    inputs = inputs.astype(jnp.int32)
    idx = inputs.reshape(-1)
    n_tokens = idx.shape[0]
    x = _sc_gather(idx, emb_table, n_tokens)
    return _tc_compute(x, centroids, inputs.shape, blk=1280)
